# Initial kernel scaffold; baseline (speedup 1.0000x reference)
import jax
import jax.numpy as jnp
from jax.experimental import pallas as pl


def kernel(x, token_expert_ids, w1, w3, w2):
    raise NotImplementedError("write your pallas kernel here")


# speedup vs baseline: 9.6588x; 9.6588x over previous
import jax
import jax.numpy as jnp
from jax.experimental import pallas as pl
from jax.experimental.pallas import tpu as pltpu

_T, _E, _D_MODEL, _D_FF = 128, 16, 768, 1536
_F_B = 768
_NF = _D_FF // _F_B


def _moe_kernel(ids_ref, x_ref, w1_ref, w3_ref, w2_ref, out_ref):
    e = pl.program_id(0)
    f = pl.program_id(1)

    mask = ids_ref[...] == e
    xm = jnp.where(mask, x_ref[...], 0.0)

    g = jax.lax.dot_general(xm, w1_ref[0], (((1,), (1,)), ((), ())),
                            preferred_element_type=jnp.float32)
    u = jax.lax.dot_general(xm, w3_ref[0], (((1,), (1,)), ((), ())),
                            preferred_element_type=jnp.float32)
    h = (g * jax.nn.sigmoid(g)) * u
    o = jax.lax.dot_general(h, w2_ref[0], (((1,), (1,)), ((), ())),
                            preferred_element_type=jnp.float32)

    @pl.when(jnp.logical_and(e == 0, f == 0))
    def _init():
        out_ref[...] = jnp.zeros_like(out_ref)

    out_ref[...] += o


def kernel(x, token_expert_ids, w1, w3, w2):
    ids = token_expert_ids.astype(jnp.int32).reshape(_T, 1)
    return pl.pallas_call(
        _moe_kernel,
        grid=(_E, _NF),
        in_specs=[
            pl.BlockSpec((_T, 1), lambda e, f: (0, 0)),
            pl.BlockSpec((_T, _D_MODEL), lambda e, f: (0, 0)),
            pl.BlockSpec((1, _F_B, _D_MODEL), lambda e, f: (e, f, 0)),
            pl.BlockSpec((1, _F_B, _D_MODEL), lambda e, f: (e, f, 0)),
            pl.BlockSpec((1, _D_MODEL, _F_B), lambda e, f: (e, 0, f)),
        ],
        out_specs=pl.BlockSpec((_T, _D_MODEL), lambda e, f: (0, 0)),
        out_shape=jax.ShapeDtypeStruct((_T, _D_MODEL), jnp.float32),
        compiler_params=pltpu.CompilerParams(
            dimension_semantics=("arbitrary", "arbitrary"),
        ),
    )(ids, x, w1, w3, w2)


# speedup vs baseline: 10.3492x; 1.0715x over previous
import jax
import jax.numpy as jnp
from jax.experimental import pallas as pl
from jax.experimental.pallas import tpu as pltpu

_T, _E, _D_MODEL, _D_FF = 128, 16, 768, 1536


def _moe_kernel(ids_ref, x_ref, w1_ref, w3_ref, w2_ref, out_ref):
    e = pl.program_id(0)

    mask = ids_ref[...] == e
    xm = jnp.where(mask, x_ref[...], 0.0)

    g = jax.lax.dot_general(xm, w1_ref[0], (((1,), (1,)), ((), ())),
                            preferred_element_type=jnp.float32)
    u = jax.lax.dot_general(xm, w3_ref[0], (((1,), (1,)), ((), ())),
                            preferred_element_type=jnp.float32)
    h = (g * jax.nn.sigmoid(g)) * u
    o = jax.lax.dot_general(h, w2_ref[0], (((1,), (1,)), ((), ())),
                            preferred_element_type=jnp.float32)

    @pl.when(e == 0)
    def _init():
        out_ref[...] = jnp.zeros_like(out_ref)

    out_ref[...] += o


def kernel(x, token_expert_ids, w1, w3, w2):
    ids = token_expert_ids.astype(jnp.int32).reshape(_T, 1)
    return pl.pallas_call(
        _moe_kernel,
        grid=(_E,),
        in_specs=[
            pl.BlockSpec((_T, 1), lambda e: (0, 0)),
            pl.BlockSpec((_T, _D_MODEL), lambda e: (0, 0)),
            pl.BlockSpec((1, _D_FF, _D_MODEL), lambda e: (e, 0, 0)),
            pl.BlockSpec((1, _D_FF, _D_MODEL), lambda e: (e, 0, 0)),
            pl.BlockSpec((1, _D_MODEL, _D_FF), lambda e: (e, 0, 0)),
        ],
        out_specs=pl.BlockSpec((_T, _D_MODEL), lambda e: (0, 0)),
        out_shape=jax.ShapeDtypeStruct((_T, _D_MODEL), jnp.float32),
        compiler_params=pltpu.CompilerParams(
            dimension_semantics=("arbitrary",),
        ),
    )(ids, x, w1, w3, w2)
